# baseline (speedup 1.0000x reference)
import jax
import jax.numpy as jnp
from jax.experimental import pallas as pl
from jax.experimental.pallas import tpu as pltpu

_K = 32
_R = 16
_N = 32768
_NS = _N // 128
_T = 5


def _cumsum_lanes(a):
    s = 1
    while s < a.shape[-1]:
        pad = jnp.zeros(a.shape[:-1] + (s,), a.dtype)
        a = a + jnp.concatenate([pad, a[..., :-s]], axis=-1)
        s *= 2
    return a


def _insert(lst, v):
    for i in range(len(lst)):
        t = jnp.minimum(lst[i], v)
        lst[i] = jnp.maximum(lst[i], v)
        v = t


def _slices(x_ref):
    for v in range(_NS):
        yield v, x_ref[:, 128 * v:128 * (v + 1)]


def _top_candidates(x_ref):
    neg = jnp.float32(-jnp.inf)
    stripes = [[jnp.full((_R, 128), neg) for _ in range(_T)]
               for _ in range(4)]
    for v, xv in _slices(x_ref):
        _insert(stripes[v % 4], xv)
    step = 4
    while step > 1:
        half = step // 2
        for a in range(half):
            for val in stripes[a + half]:
                _insert(stripes[a], val)
        step = half
    return stripes[0]


def _bisect(cands, iters):
    lo = jnp.min(cands[0], axis=-1, keepdims=True)
    hi = jnp.max(cands[0], axis=-1, keepdims=True)
    for _ in range(iters):
        t = lo + (hi - lo) * 0.5
        cc = (cands[0] > t).astype(jnp.int32)
        for m in cands[1:]:
            cc = cc + (m > t).astype(jnp.int32)
        ok = jnp.sum(cc, axis=-1, keepdims=True) >= _K
        lo = jnp.where(ok, t, lo)
        hi = jnp.where(ok, hi, t)
    return lo


def _probe(x_ref, t):
    pos = jnp.float32(jnp.inf)
    cnts = [jnp.zeros((_R, 128), jnp.int32) for _ in range(8)]
    mns = [jnp.full((_R, 128), pos) for _ in range(8)]
    for v, xv in _slices(x_ref):
        a = v % 8
        gt = xv > t
        cnts[a] = cnts[a] + gt.astype(jnp.int32)
        mns[a] = jnp.minimum(mns[a], jnp.where(gt, xv, pos))
    cnt, mn = cnts[0], mns[0]
    for a in range(1, 8):
        cnt = cnt + cnts[a]
        mn = jnp.minimum(mn, mns[a])
    return (jnp.sum(cnt, axis=-1, keepdims=True),
            jnp.min(mn, axis=-1, keepdims=True))


def _mask_write(x_ref, o_ref, tau):
    cgs = [jnp.zeros((_R, 128), jnp.int32) for _ in range(8)]
    ces = [jnp.zeros((_R, 128), jnp.int32) for _ in range(8)]
    for v, xv in _slices(x_ref):
        a = v % 8
        gt = xv > tau
        eq = xv == tau
        o_ref[:, 128 * v:128 * (v + 1)] = jnp.where(gt | eq, xv, 0.0)
        cgs[a] = cgs[a] + gt.astype(jnp.int32)
        ces[a] = ces[a] + eq.astype(jnp.int32)
    cg, ce = cgs[0], ces[0]
    for a in range(1, 8):
        cg = cg + cgs[a]
        ce = ce + ces[a]
    return (jnp.sum(cg, axis=-1, keepdims=True),
            jnp.sum(ce, axis=-1, keepdims=True))


def _body_fast(x_ref, o_ref, tau_o, c_o, ceq_o):
    cands = _top_candidates(x_ref)
    tau = _bisect(cands, 24)
    for _ in range(2):
        c0, nxt0 = _probe(x_ref, tau)
        tau = jnp.where(c0 >= _K, nxt0, tau)
    c, ceq = _mask_write(x_ref, o_ref, tau)
    tau_o[...] = tau
    c_o[...] = c
    ceq_o[...] = ceq


def _body_full(x_ref, o_ref, tau_s, c_s, ceq_s):
    cands = _top_candidates(x_ref)
    tau0 = _bisect(cands, 24)
    c0, nxt0 = _probe(x_ref, tau0)

    def cond(carry):
        _t, c, _n = carry
        return jnp.any(c >= _K)

    def bodyw(carry):
        t, c, nxt = carry
        newt = jnp.where(c >= _K, nxt, t)
        newc, newn = _probe(x_ref, newt)
        return newt, newc, newn

    tau, c, _ = jax.lax.while_loop(cond, bodyw, (tau0, c0, nxt0))
    _c2, ceq = _mask_write(x_ref, o_ref, tau)
    tau_s[...] = tau
    c_s[...] = c
    ceq_s[...] = ceq

    r = _K - c

    @pl.when(jnp.any(ceq > r))
    def _():
        base = jnp.zeros((_R, 1), jnp.int32)
        for v, xv in _slices(x_ref):
            eqi = (xv == tau).astype(jnp.int32)
            pref = _cumsum_lanes(eqi) - eqi + base
            keep = (xv > tau) | ((eqi > 0) & (pref < r))
            o_ref[:, 128 * v:128 * (v + 1)] = jnp.where(keep, xv, 0.0)
            base = base + jnp.sum(eqi, axis=-1, keepdims=True)


def _call_fast(x):
    grid = x.shape[0] // _R
    return pl.pallas_call(
        _body_fast,
        grid=(grid,),
        in_specs=[pl.BlockSpec((_R, _N), lambda i: (i, 0))],
        out_specs=[
            pl.BlockSpec((_R, _N), lambda i: (i, 0)),
            pl.BlockSpec((_R, 1), lambda i: (i, 0)),
            pl.BlockSpec((_R, 1), lambda i: (i, 0)),
            pl.BlockSpec((_R, 1), lambda i: (i, 0)),
        ],
        out_shape=[
            jax.ShapeDtypeStruct(x.shape, x.dtype),
            jax.ShapeDtypeStruct((x.shape[0], 1), jnp.float32),
            jax.ShapeDtypeStruct((x.shape[0], 1), jnp.int32),
            jax.ShapeDtypeStruct((x.shape[0], 1), jnp.int32),
        ],
        compiler_params=pltpu.CompilerParams(
            dimension_semantics=("parallel",)
        ),
    )(x)


def _call_full(x):
    grid = x.shape[0] // _R
    return pl.pallas_call(
        _body_full,
        grid=(grid,),
        in_specs=[pl.BlockSpec((_R, _N), lambda i: (i, 0))],
        out_specs=pl.BlockSpec((_R, _N), lambda i: (i, 0)),
        out_shape=jax.ShapeDtypeStruct(x.shape, x.dtype),
        scratch_shapes=[
            pltpu.VMEM((_R, 1), jnp.float32),
            pltpu.VMEM((_R, 1), jnp.int32),
            pltpu.VMEM((_R, 1), jnp.int32),
        ],
        compiler_params=pltpu.CompilerParams(
            dimension_semantics=("arbitrary",)
        ),
    )(x)


@jax.jit
def kernel(x):
    y, _tau, c, ceq = _call_fast(x)
    bad = jnp.any(c >= _K) | jnp.any(ceq != _K - c)
    return jax.lax.cond(bad,
                        lambda ops: _call_full(ops[0]),
                        lambda ops: ops[1],
                        (x, y))
